# baseline (speedup 1.0000x reference)
import functools

import jax
import jax.numpy as jnp
from jax import lax
from jax.experimental import pallas as pl
from jax.experimental.pallas import tpu as pltpu
from jax.experimental.pallas import tpu_sc as plsc

B, S, V, D, C = 32, 512, 862, 1024, 64
VPAD = 1024
NW = 32
ROWS_PER_W = VPAD // NW
LANES = 16


RB = 4
SV128 = S * V // 128


def _tc_reduce_body(x_ref, out_ref):
    i = pl.program_id(0)
    xb = x_ref[...].astype(jnp.bfloat16).astype(jnp.float32)
    partial = (xb[0] + xb[1]) + (xb[2] + xb[3])

    @pl.when(i == 0)
    def _():
        out_ref[...] = partial

    @pl.when(i > 0)
    def _():
        out_ref[...] = out_ref[...] + partial


def _tc_reduce(x3):
    return pl.pallas_call(
        _tc_reduce_body,
        grid=(B // RB,),
        in_specs=[pl.BlockSpec((RB, SV128, 128), lambda i: (i, 0, 0))],
        out_specs=pl.BlockSpec((SV128, 128), lambda i: (0, 0)),
        out_shape=jax.ShapeDtypeStruct((SV128, 128), jnp.float32),
    )(x3)


def _tc_scores_body(m_ref, wt_ref, b_ref, cen_ref, out_ref):
    m = m_ref[...] * (1.0 / B)
    mh = m.astype(jnp.bfloat16)
    r1 = m - mh.astype(jnp.float32)
    ml = r1.astype(jnp.bfloat16)
    ml2 = (r1 - ml.astype(jnp.float32)).astype(jnp.bfloat16)
    wt = wt_ref[...]

    def dotb(a):
        return lax.dot_general(
            a, wt, dimension_numbers=(((0,), (0,)), ((), ())),
            preferred_element_type=jnp.float32)

    emb = ((dotb(mh) + dotb(ml)) + dotb(ml2)) + b_ref[...]
    en = jnp.sqrt(jnp.sum(emb * emb, axis=1, keepdims=True))
    emb = emb / jnp.maximum(en, 1e-12)
    cen = cen_ref[...]
    norm = jnp.sqrt(jnp.sum(cen * cen, axis=1, keepdims=True))
    cn = cen / jnp.maximum(norm, 1e-12)
    scores = lax.dot_general(
        emb, cn,
        dimension_numbers=(((1,), (1,)), ((), ())),
        preferred_element_type=jnp.float32,
    )
    out_ref[...] = jnp.concatenate(
        [scores, jnp.zeros((VPAD - V, C), jnp.float32)], axis=0)


def _tc_scores(msum, wtb, b2, cen):
    return pl.pallas_call(
        _tc_scores_body,
        in_specs=[
            pl.BlockSpec((S, V), lambda: (0, 0)),
            pl.BlockSpec((S, D), lambda: (0, 0)),
            pl.BlockSpec((1, D), lambda: (0, 0)),
            pl.BlockSpec((C, D), lambda: (0, 0)),
        ],
        out_specs=pl.BlockSpec((VPAD, C), lambda: (0, 0)),
        out_shape=jax.ShapeDtypeStruct((VPAD, C), jnp.float32),
    )(msum, wtb, b2, cen)


def _argmax_merge(v, i, w, j):
    keep = (v > w) | ((v == w) & (i < j))
    return jnp.where(keep, v, w), jnp.where(keep, i, j)


def _sc_body(scores_hbm, out_hbm, sc_v, out_v):
    wid = lax.axis_index("s") * 2 + lax.axis_index("c")
    base = wid * ROWS_PER_W
    pltpu.sync_copy(scores_hbm.at[pl.ds(base, ROWS_PER_W)], sc_v)
    iota = lax.iota(jnp.int32, LANES)
    perms = [jnp.bitwise_xor(iota, d) for d in (8, 4, 2, 1)]
    for r in range(ROWS_PER_W):
        v, i = None, None
        for c in range(C // LANES):
            w = sc_v[r, pl.ds(c * LANES, LANES)]
            j = iota + c * LANES
            v, i = (w, j) if v is None else _argmax_merge(v, i, w, j)
        for p in perms:
            v, i = _argmax_merge(
                v, i,
                v.at[p].get(mode="promise_in_bounds", unique_indices=True),
                i.at[p].get(mode="promise_in_bounds", unique_indices=True))
        for c in range(C // LANES):
            out_v[r, pl.ds(c * LANES, LANES)] = jnp.where(
                iota + c * LANES == i, 1.0, 0.0).astype(jnp.float32)
    pltpu.sync_copy(out_v, out_hbm.at[pl.ds(base, ROWS_PER_W)])


def _sc_assign(scores):
    mesh = plsc.VectorSubcoreMesh(core_axis_name="c", subcore_axis_name="s",
                                  num_cores=2, num_subcores=16)
    return pl.kernel(
        _sc_body,
        out_type=jax.ShapeDtypeStruct((VPAD, C), jnp.float32),
        mesh=mesh,
        scratch_types=[
            pltpu.VMEM((ROWS_PER_W, C), jnp.float32),
            pltpu.VMEM((ROWS_PER_W, C), jnp.float32),
        ],
    )(scores)


def _scores_pipeline(x, W, b, centroids):
    x3 = x.reshape(B, SV128, 128)
    msum = _tc_reduce(x3)
    m = msum.reshape(S, V)
    return _tc_scores(m, W.T.astype(jnp.bfloat16), b.reshape(1, D),
                      centroids)


@jax.jit
def kernel(x, W, b, centroids):
    scores = _scores_pipeline(x, W, b, centroids)
    assignments = _sc_assign(scores)
    return assignments[:V]


# speedup vs baseline: 1.4882x; 1.4882x over previous
import functools

import jax
import jax.numpy as jnp
from jax import lax
from jax.experimental import pallas as pl
from jax.experimental.pallas import tpu as pltpu
from jax.experimental.pallas import tpu_sc as plsc

B, S, V, D, C = 32, 512, 862, 1024, 64
VPAD = 1024
NW = 32
ROWS_PER_W = VPAD // NW
LANES = 16


RB = 4


def _tc_reduce_body(x_ref, out_ref):
    i = pl.program_id(0)
    xb = x_ref[...].astype(jnp.bfloat16).astype(jnp.float32)
    partial = (xb[0] + xb[1]) + (xb[2] + xb[3])

    @pl.when(i == 0)
    def _():
        out_ref[...] = partial

    @pl.when(i > 0)
    def _():
        out_ref[...] = out_ref[...] + partial


def _tc_reduce(x3):
    return pl.pallas_call(
        _tc_reduce_body,
        grid=(B // RB,),
        in_specs=[pl.BlockSpec((RB, S, V), lambda i: (i, 0, 0))],
        out_specs=pl.BlockSpec((S, V), lambda i: (0, 0)),
        out_shape=jax.ShapeDtypeStruct((S, V), jnp.float32),
    )(x3)


def _tc_scores_body(m_ref, wt_ref, b_ref, cen_ref, out_ref):
    m = m_ref[...] * (1.0 / B)
    mh = m.astype(jnp.bfloat16)
    r1 = m - mh.astype(jnp.float32)
    ml = r1.astype(jnp.bfloat16)
    ml2 = (r1 - ml.astype(jnp.float32)).astype(jnp.bfloat16)
    wt = wt_ref[...]

    def dotb(a):
        return lax.dot_general(
            a, wt, dimension_numbers=(((0,), (0,)), ((), ())),
            preferred_element_type=jnp.float32)

    emb = ((dotb(mh) + dotb(ml)) + dotb(ml2)) + b_ref[...]
    en = jnp.sqrt(jnp.sum(emb * emb, axis=1, keepdims=True))
    emb = emb / jnp.maximum(en, 1e-12)
    cen = cen_ref[...]
    norm = jnp.sqrt(jnp.sum(cen * cen, axis=1, keepdims=True))
    cn = cen / jnp.maximum(norm, 1e-12)
    scores = lax.dot_general(
        emb, cn,
        dimension_numbers=(((1,), (1,)), ((), ())),
        preferred_element_type=jnp.float32,
    )
    out_ref[...] = jnp.concatenate(
        [scores, jnp.zeros((VPAD - V, C), jnp.float32)], axis=0)


def _tc_scores(msum, wtb, b2, cen):
    return pl.pallas_call(
        _tc_scores_body,
        in_specs=[
            pl.BlockSpec((S, V), lambda: (0, 0)),
            pl.BlockSpec((S, D), lambda: (0, 0)),
            pl.BlockSpec((1, D), lambda: (0, 0)),
            pl.BlockSpec((C, D), lambda: (0, 0)),
        ],
        out_specs=pl.BlockSpec((VPAD, C), lambda: (0, 0)),
        out_shape=jax.ShapeDtypeStruct((VPAD, C), jnp.float32),
    )(msum, wtb, b2, cen)


def _argmax_merge(v, i, w, j):
    keep = (v > w) | ((v == w) & (i < j))
    return jnp.where(keep, v, w), jnp.where(keep, i, j)


def _sc_body(scores_hbm, out_hbm, sc_v, out_v):
    wid = lax.axis_index("s") * 2 + lax.axis_index("c")
    base = wid * ROWS_PER_W
    pltpu.sync_copy(scores_hbm.at[pl.ds(base, ROWS_PER_W)], sc_v)
    iota = lax.iota(jnp.int32, LANES)
    perms = [jnp.bitwise_xor(iota, d) for d in (8, 4, 2, 1)]
    for r in range(ROWS_PER_W):
        v, i = None, None
        for c in range(C // LANES):
            w = sc_v[r, pl.ds(c * LANES, LANES)]
            j = iota + c * LANES
            v, i = (w, j) if v is None else _argmax_merge(v, i, w, j)
        for p in perms:
            v, i = _argmax_merge(
                v, i,
                v.at[p].get(mode="promise_in_bounds", unique_indices=True),
                i.at[p].get(mode="promise_in_bounds", unique_indices=True))
        for c in range(C // LANES):
            out_v[r, pl.ds(c * LANES, LANES)] = jnp.where(
                iota + c * LANES == i, 1.0, 0.0).astype(jnp.float32)
    pltpu.sync_copy(out_v, out_hbm.at[pl.ds(base, ROWS_PER_W)])


def _sc_assign(scores):
    mesh = plsc.VectorSubcoreMesh(core_axis_name="c", subcore_axis_name="s",
                                  num_cores=2, num_subcores=16)
    return pl.kernel(
        _sc_body,
        out_type=jax.ShapeDtypeStruct((VPAD, C), jnp.float32),
        mesh=mesh,
        scratch_types=[
            pltpu.VMEM((ROWS_PER_W, C), jnp.float32),
            pltpu.VMEM((ROWS_PER_W, C), jnp.float32),
        ],
    )(scores)


def _scores_pipeline(x, W, b, centroids):
    msum = _tc_reduce(x)
    return _tc_scores(msum, W.T.astype(jnp.bfloat16), b.reshape(1, D),
                      centroids)


@jax.jit
def kernel(x, W, b, centroids):
    scores = _scores_pipeline(x, W, b, centroids)
    assignments = _sc_assign(scores)
    return assignments[:V]
